# Initial kernel scaffold; baseline (speedup 1.0000x reference)
import jax
import jax.numpy as jnp
from jax.experimental import pallas as pl


def kernel(x_user, x_place, ei_uu, ea_uu, ei_up, ea_up, ei_pu, ea_pu, params):
    raise NotImplementedError("write your pallas kernel here")


# speedup vs baseline: 1.0230x; 1.0230x over previous
import functools

import jax
import jax.numpy as jnp
from jax.experimental import pallas as pl
from jax.experimental.pallas import tpu as pltpu

N_USER = 50000
N_PLACE = 50000
D = 128
NUM_LAYERS = 2

_BLK = 1000


def _combine_body(aggr_a_ref, aggr_b_ref, x_ref, res_ref, wa_ref, wb_ref,
                  wr_ref, bias_ref, g_ref, b_ref, o_ref, *, with_b, with_res):
    x = x_ref[...]
    acc = jnp.dot(aggr_a_ref[...], wa_ref[...],
                  preferred_element_type=jnp.float32)
    if with_b:
        acc = acc + jnp.dot(aggr_b_ref[...], wb_ref[...],
                            preferred_element_type=jnp.float32)
    acc = acc + jnp.dot(x, wr_ref[...], preferred_element_type=jnp.float32)
    acc = acc + bias_ref[...]
    acc = jnp.maximum(acc, 0.0)
    mu = jnp.mean(acc, axis=-1, keepdims=True)
    var = jnp.mean((acc - mu) ** 2, axis=-1, keepdims=True)
    y = (acc - mu) * jax.lax.rsqrt(var + 1e-5) * g_ref[...] + b_ref[...]
    if with_res:
        y = y + res_ref[...]
    o_ref[...] = y


def _combine(aggr_a, aggr_b, x, res, wa, wb, wr, bias, g, b,
             with_b, with_res):
    n = x.shape[0]
    grid = (n // _BLK,)
    row_spec = pl.BlockSpec((_BLK, D), lambda i: (i, 0))
    mat_spec = pl.BlockSpec((D, D), lambda i: (0, 0))
    vec_spec = pl.BlockSpec((1, D), lambda i: (0, 0))
    return pl.pallas_call(
        functools.partial(_combine_body, with_b=with_b, with_res=with_res),
        grid=grid,
        in_specs=[row_spec, row_spec, row_spec, row_spec, mat_spec, mat_spec,
                  mat_spec, vec_spec, vec_spec, vec_spec],
        out_specs=row_spec,
        out_shape=jax.ShapeDtypeStruct((n, D), jnp.float32),
    )(aggr_a, aggr_b, x, res, wa, wb, wr, bias, g, b)


def _edge_weight(ea, we, be):
    ew = jax.nn.relu(ea @ we + be)
    return ew.mean(axis=1)


def _segment_mean(msg, dst, num_segments):
    s = jax.ops.segment_sum(msg, dst, num_segments=num_segments)
    c = jax.ops.segment_sum(jnp.ones((msg.shape[0], 1), msg.dtype), dst,
                            num_segments=num_segments)
    return s / jnp.clip(c, 1.0, None)


def kernel(x_user, x_place, ei_uu, ea_uu, ei_up, ea_up, ei_pu, ea_pu, params):
    h_u, h_p = x_user, x_place
    for l in range(NUM_LAYERS):
        lp = params['layer%d' % l]
        p_uu, p_pu, p_up = lp['uu'], lp['pu'], lp['up']

        w_uu = _edge_weight(ea_uu, p_uu['We'], p_uu['be'])
        w_pu = _edge_weight(ea_pu, p_pu['We'], p_pu['be'])
        w_up = _edge_weight(ea_up, p_up['We'], p_up['be'])

        aggr_uu = _segment_mean(h_u[ei_uu[0]] * w_uu[:, None], ei_uu[1], N_USER)
        aggr_pu = _segment_mean(h_p[ei_pu[0]] * w_pu[:, None], ei_pu[1], N_USER)
        aggr_up = _segment_mean(h_u[ei_up[0]] * w_up[:, None], ei_up[1], N_PLACE)

        bias_u = (p_uu['bl'] + p_uu['br'] + p_pu['bl'] + p_pu['br'])[None, :]
        bias_p = (p_up['bl'] + p_up['br'])[None, :]
        wr_u = p_uu['Wr'] + p_pu['Wr']

        out_u = _combine(aggr_uu, aggr_pu, h_u, h_u, p_uu['Wl'], p_pu['Wl'],
                         wr_u, bias_u, lp['ln_u_g'][None, :], lp['ln_u_b'][None, :],
                         with_b=True, with_res=(l > 0))
        out_p = _combine(aggr_up, aggr_up, h_p, h_p, p_up['Wl'], p_up['Wl'],
                         p_up['Wr'], bias_p, lp['ln_p_g'][None, :], lp['ln_p_b'][None, :],
                         with_b=False, with_res=(l > 0))
        h_u, h_p = out_u, out_p
    return h_u, h_p
